# baseline (speedup 1.0000x reference)
import jax
import jax.numpy as jnp
from jax import lax
from jax.experimental import pallas as pl
from jax.experimental.pallas import tpu as pltpu
from jax.experimental.pallas import tpu_sc as plsc

NC = 2
NS = 16
L = 16
NW = NC * NS
CH = 40


def _vrsqrt(v):
    y = plsc.bitcast(jnp.int32(0x5F3759DF) - (plsc.bitcast(v, jnp.int32) >> 1),
                     jnp.float32)
    h = v * 0.5
    for _ in range(3):
        y = y * (1.5 - h * y * y)
    return y


def _deg_body(src_hbm, dst_hbm, feat_hbm, nf_hbm, nd_hbm,
              idx_v, hist, tmp, red, nrm, bbuf, slab):
    ept = src_hbm.shape[0] // NS
    npad = nd_hbm.shape[0]
    n = feat_hbm.shape[0]
    rpt = npad // NS
    d = nd_hbm.shape[1]
    vpr = d // L
    cid = lax.axis_index("c")
    sid = lax.axis_index("s")

    @pl.when(cid == 0)
    def _():
        pltpu.sync_copy(src_hbm.at[pl.ds(sid * ept, ept)], idx_v)

    @pl.when(cid == 1)
    def _():
        pltpu.sync_copy(dst_hbm.at[pl.ds(sid * ept, ept)], idx_v)

    zero = jnp.zeros((L,), jnp.float32)
    UZ = 8

    def zloop(i, c):
        for u in range(UZ):
            hist[pl.ds((i * UZ + u) * L, L)] = zero
        return c

    lax.fori_loop(0, npad // (L * UZ), zloop, 0)
    ones = jnp.ones((L,), jnp.float32)
    UE = 5

    def eloop(i, c):
        for u in range(UE):
            plsc.addupdate_scatter(hist, [idx_v[pl.ds((i * UE + u) * L, L)]],
                                   ones)
        return c

    lax.fori_loop(0, ept // (L * UE), eloop, 0)

    pltpu.sync_copy(hist, slab.at[sid])
    plsc.subcore_barrier()

    def rzero(i, c):
        for u in range(UZ):
            red[pl.ds((i * UZ + u) * L, L)] = zero
        return c

    lax.fori_loop(0, rpt // (L * UZ), rzero, 0)

    def radd(j, c):
        pltpu.sync_copy(slab.at[j, pl.ds(sid * rpt, rpt)], tmp)

        def racc(i, cc):
            for u in range(UE):
                sl = pl.ds((i * UE + u) * L, L)
                red[sl] += tmp[sl]
            return cc

        return lax.fori_loop(0, rpt // (L * UE), racc, c)

    lax.fori_loop(0, NS, radd, 0)

    def rnorm(i, c):
        for u in range(UE):
            sl = pl.ds((i * UE + u) * L, L)
            nrm[sl] = _vrsqrt(jnp.maximum(red[sl], 1.0))
        return c

    lax.fori_loop(0, rpt // (L * UE), rnorm, 0)

    nb = rpt // CH
    my_chunks = jnp.maximum(jnp.minimum((n - sid * rpt) // CH, nb), 0)

    @pl.when(cid == 0)
    def _():
        def sloop(b, c):
            base = sid * rpt + b * CH
            pltpu.sync_copy(feat_hbm.at[pl.ds(base, CH)], bbuf)

            def rowscale(i, cc):
                vals = nrm[pl.ds(b * CH + i * L, L)]
                for kk in range(L):
                    for k in range(vpr):
                        sl = pl.ds(k * L, L)
                        bbuf[i * L + kk, sl] = bbuf[i * L + kk, sl] * vals[kk]
                return cc

            lax.fori_loop(0, CH // L, rowscale, c)
            pltpu.sync_copy(bbuf, nf_hbm.at[pl.ds(base, CH)])
            return c

        lax.fori_loop(0, my_chunks, sloop, 0)

    @pl.when(cid == 1)
    def _():
        def bloop(b, c):
            def rowfill(i, cc):
                vals = nrm[pl.ds(b * CH + i * L, L)]
                for kk in range(L):
                    row = jnp.full((L,), vals[kk], jnp.float32)
                    for k in range(vpr):
                        bbuf[i * L + kk, pl.ds(k * L, L)] = row
                return cc

            lax.fori_loop(0, CH // L, rowfill, c)
            pltpu.sync_copy(bbuf, nd_hbm.at[pl.ds(sid * rpt + b * CH, CH)])
            return c

        lax.fori_loop(0, nb, bloop, 0)


def _degree_norms(src, dst, features, npad):
    e = src.shape[0]
    n, d = features.shape
    ept = e // NS
    rpt = npad // NS
    f = pl.kernel(
        _deg_body,
        out_type=[jax.ShapeDtypeStruct((n, d), jnp.float32),
                  jax.ShapeDtypeStruct((npad, d), jnp.float32)],
        mesh=plsc.VectorSubcoreMesh(core_axis_name="c", subcore_axis_name="s"),
        compiler_params=pltpu.CompilerParams(needs_layout_passes=False),
        scratch_types=[
            pltpu.VMEM((ept,), jnp.int32),
            pltpu.VMEM((npad,), jnp.float32),
            pltpu.VMEM((rpt,), jnp.float32),
            pltpu.VMEM((rpt,), jnp.float32),
            pltpu.VMEM((rpt,), jnp.float32),
            pltpu.VMEM((CH, d), jnp.float32),
            pltpu.VMEM_SHARED((NS, npad), jnp.float32),
        ],
    )
    return f(src, dst, features)


def _mp_body(nf_hbm, srcr_hbm, dstr_hbm, acc_hbm,
             src_v, dst_c, rows_v, gs0, gs1, gs2, ss0, ss1, ss2, acc_s):
    gsems = (gs0, gs1, gs2)
    ssems = (ss0, ss1, ss2)
    nch = src_v.shape[0]
    npad = acc_s.shape[0]
    rows_per_tile = npad // NS
    zr = rows_v.shape[1]
    cid = lax.axis_index("c")
    sid = lax.axis_index("s")
    wid = sid * NC + cid
    pltpu.sync_copy(srcr_hbm.at[wid], src_v)

    zero = jnp.zeros((L,), jnp.float32)
    vecs_per_row = rows_v.shape[2] // L

    def zloop(i, c):
        rows_v[0, i // vecs_per_row, pl.ds((i % vecs_per_row) * L, L)] = zero
        return c

    lax.fori_loop(0, zr * vecs_per_row, zloop, 0)
    for k in range(rows_per_tile // zr):
        pltpu.sync_copy(rows_v.at[0],
                        acc_s.at[pl.ds(sid * rows_per_tile + k * zr, zr)])
    plsc.subcore_barrier()

    NB = 3

    def start(j, u):
        pltpu.async_copy(nf_hbm.at[src_v.at[j]], rows_v.at[u], gsems[u])
        pltpu.async_copy(dstr_hbm.at[wid, j, 0], dst_c.at[u], gsems[u])

    def gwait(j, u):
        pltpu.make_async_copy(nf_hbm.at[src_v.at[j]], rows_v.at[u],
                              gsems[u]).wait()
        pltpu.make_async_copy(dstr_hbm.at[wid, j, 0], dst_c.at[u],
                              gsems[u]).wait()

    def sissue(u):
        pltpu.async_copy(rows_v.at[u], acc_s.at[dst_c.at[u]], ssems[u],
                         add=True)

    def swait(u):
        pltpu.make_async_copy(rows_v.at[u], acc_s.at[dst_c.at[u]],
                              ssems[u]).wait()

    def step(j, u, first):
        gwait(j, u)
        sissue(u)
        if first:
            @pl.when(j > 0)
            def _():
                swait((u + 2) % NB)
        else:
            swait((u + 2) % NB)

    start(0, 0)
    start(1, 1)

    def triple(g, c):
        j = 3 * g
        step(j, 0, True)
        start(j + 2, 2)
        step(j + 1, 1, False)
        start(j + 3, 0)
        step(j + 2, 2, False)

        @pl.when(j + 4 < nch)
        def _():
            start(j + 4, 1)

        return c

    lax.fori_loop(0, nch // 3, triple, 0)
    for k in range(3 * (nch // 3), nch):
        step(k, k % NB, False)
    swait((nch - 1) % NB)
    plsc.subcore_barrier()
    for k in range(rows_per_tile // zr):
        sl = pl.ds(sid * rows_per_tile + k * zr, zr)
        pltpu.sync_copy(acc_s.at[sl], acc_hbm.at[cid, sl])


def _message_pass(nf, srcr, dstr, npad):
    n, d = nf.shape
    nch = srcr.shape[1]
    assert nch >= 5 and nch % 3 != 0
    f = pl.kernel(
        _mp_body,
        out_type=jax.ShapeDtypeStruct((NC, npad, d), jnp.float32),
        mesh=plsc.VectorSubcoreMesh(core_axis_name="c", subcore_axis_name="s"),
        compiler_params=pltpu.CompilerParams(needs_layout_passes=False),
        scratch_types=[
            pltpu.VMEM((nch, CH), jnp.int32),
            pltpu.VMEM((3, CH), jnp.int32),
            pltpu.VMEM((3, CH, d), jnp.float32),
            pltpu.SemaphoreType.DMA,
            pltpu.SemaphoreType.DMA,
            pltpu.SemaphoreType.DMA,
            pltpu.SemaphoreType.DMA,
            pltpu.SemaphoreType.DMA,
            pltpu.SemaphoreType.DMA,
            pltpu.VMEM_SHARED((npad, d), jnp.float32),
        ],
    )
    return f(nf, srcr, dstr)


def _final_body(acc_ref, nd_ref, out_ref):
    out_ref[...] = (acc_ref[0] + acc_ref[1]) * nd_ref[...]


def kernel(features, edge_index):
    n, d = features.shape
    e = edge_index.shape[1]
    assert e % (NW * CH) == 0 and d % L == 0
    src = edge_index[0].astype(jnp.int32)
    dst = edge_index[1].astype(jnp.int32)

    rpt = ((n + NS - 1) // NS + 2 * CH - 1) // (2 * CH) * (2 * CH)
    npad = NS * rpt

    nf, norm_dst = _degree_norms(src, dst, features, npad)

    r = 1000
    ep = e // NW
    srcr = src.reshape(NW, ep // CH, CH)
    dstr = dst.reshape(NW, ep // CH, 1, CH)
    acc = _message_pass(nf, srcr, dstr, npad)

    out = pl.pallas_call(
        _final_body,
        grid=(n // r,),
        in_specs=[
            pl.BlockSpec((NC, r, d), lambda i: (0, i, 0)),
            pl.BlockSpec((r, d), lambda i: (i, 0)),
        ],
        out_specs=pl.BlockSpec((r, d), lambda i: (i, 0)),
        out_shape=jax.ShapeDtypeStruct((n, d), jnp.float32),
    )(acc, norm_dst)
    return out


# speedup vs baseline: 1.1263x; 1.1263x over previous
import jax
import jax.numpy as jnp
from jax import lax
from jax.experimental import pallas as pl
from jax.experimental.pallas import tpu as pltpu
from jax.experimental.pallas import tpu_sc as plsc

NC = 2
NS = 16
L = 16
NW = NC * NS
CH = 80


def _vrsqrt(v):
    y = plsc.bitcast(jnp.int32(0x5F3759DF) - (plsc.bitcast(v, jnp.int32) >> 1),
                     jnp.float32)
    h = v * 0.5
    for _ in range(3):
        y = y * (1.5 - h * y * y)
    return y


def _deg_body(src_hbm, dst_hbm, feat_hbm, nf_hbm, nd_hbm,
              idx_v, hist, tmp, red, nrm, bbuf, slab):
    ept = src_hbm.shape[0] // NS
    npad = nd_hbm.shape[0]
    n = feat_hbm.shape[0]
    rpt = npad // NS
    d = nd_hbm.shape[1]
    vpr = d // L
    cid = lax.axis_index("c")
    sid = lax.axis_index("s")

    @pl.when(cid == 0)
    def _():
        pltpu.sync_copy(src_hbm.at[pl.ds(sid * ept, ept)], idx_v)

    @pl.when(cid == 1)
    def _():
        pltpu.sync_copy(dst_hbm.at[pl.ds(sid * ept, ept)], idx_v)

    zero = jnp.zeros((L,), jnp.float32)
    UZ = 8

    def zloop(i, c):
        for u in range(UZ):
            hist[pl.ds((i * UZ + u) * L, L)] = zero
        return c

    lax.fori_loop(0, npad // (L * UZ), zloop, 0)
    ones = jnp.ones((L,), jnp.float32)
    UE = 5

    def eloop(i, c):
        for u in range(UE):
            plsc.addupdate_scatter(hist, [idx_v[pl.ds((i * UE + u) * L, L)]],
                                   ones)
        return c

    lax.fori_loop(0, ept // (L * UE), eloop, 0)

    pltpu.sync_copy(hist, slab.at[sid])
    plsc.subcore_barrier()

    def rzero(i, c):
        for u in range(UZ):
            red[pl.ds((i * UZ + u) * L, L)] = zero
        return c

    lax.fori_loop(0, rpt // (L * UZ), rzero, 0)

    def radd(j, c):
        pltpu.sync_copy(slab.at[j, pl.ds(sid * rpt, rpt)], tmp)

        def racc(i, cc):
            for u in range(UE):
                sl = pl.ds((i * UE + u) * L, L)
                red[sl] += tmp[sl]
            return cc

        return lax.fori_loop(0, rpt // (L * UE), racc, c)

    lax.fori_loop(0, NS, radd, 0)

    def rnorm(i, c):
        for u in range(UE):
            sl = pl.ds((i * UE + u) * L, L)
            nrm[sl] = _vrsqrt(jnp.maximum(red[sl], 1.0))
        return c

    lax.fori_loop(0, rpt // (L * UE), rnorm, 0)

    nb = rpt // CH
    my_chunks = jnp.maximum(jnp.minimum((n - sid * rpt) // CH, nb), 0)

    @pl.when(cid == 0)
    def _():
        def sloop(b, c):
            base = sid * rpt + b * CH
            pltpu.sync_copy(feat_hbm.at[pl.ds(base, CH)], bbuf)

            def rowscale(i, cc):
                vals = nrm[pl.ds(b * CH + i * L, L)]
                for kk in range(L):
                    for k in range(vpr):
                        sl = pl.ds(k * L, L)
                        bbuf[i * L + kk, sl] = bbuf[i * L + kk, sl] * vals[kk]
                return cc

            lax.fori_loop(0, CH // L, rowscale, c)
            pltpu.sync_copy(bbuf, nf_hbm.at[pl.ds(base, CH)])
            return c

        lax.fori_loop(0, my_chunks, sloop, 0)

    @pl.when(cid == 1)
    def _():
        def bloop(b, c):
            def rowfill(i, cc):
                vals = nrm[pl.ds(b * CH + i * L, L)]
                for kk in range(L):
                    row = jnp.full((L,), vals[kk], jnp.float32)
                    for k in range(vpr):
                        bbuf[i * L + kk, pl.ds(k * L, L)] = row
                return cc

            lax.fori_loop(0, CH // L, rowfill, c)
            pltpu.sync_copy(bbuf, nd_hbm.at[pl.ds(sid * rpt + b * CH, CH)])
            return c

        lax.fori_loop(0, nb, bloop, 0)


def _degree_norms(src, dst, features, npad):
    e = src.shape[0]
    n, d = features.shape
    ept = e // NS
    rpt = npad // NS
    f = pl.kernel(
        _deg_body,
        out_type=[jax.ShapeDtypeStruct((n, d), jnp.float32),
                  jax.ShapeDtypeStruct((npad, d), jnp.float32)],
        mesh=plsc.VectorSubcoreMesh(core_axis_name="c", subcore_axis_name="s"),
        compiler_params=pltpu.CompilerParams(needs_layout_passes=False),
        scratch_types=[
            pltpu.VMEM((ept,), jnp.int32),
            pltpu.VMEM((npad,), jnp.float32),
            pltpu.VMEM((rpt,), jnp.float32),
            pltpu.VMEM((rpt,), jnp.float32),
            pltpu.VMEM((rpt,), jnp.float32),
            pltpu.VMEM((CH, d), jnp.float32),
            pltpu.VMEM_SHARED((NS, npad), jnp.float32),
        ],
    )
    return f(src, dst, features)


def _mp_body(nf_hbm, srcr_hbm, dstr_hbm, acc_hbm,
             src_v, dst_c, rows_v, gsem0, gsem1, acc_s):
    nch = src_v.shape[0]
    npad = acc_s.shape[0]
    rows_per_tile = npad // NS
    zr = rows_v.shape[1]
    cid = lax.axis_index("c")
    sid = lax.axis_index("s")
    wid = sid * NC + cid
    pltpu.sync_copy(srcr_hbm.at[wid], src_v)

    zero = jnp.zeros((L,), jnp.float32)
    vecs_per_row = rows_v.shape[2] // L

    def zloop(i, c):
        rows_v[0, i // vecs_per_row, pl.ds((i % vecs_per_row) * L, L)] = zero
        return c

    lax.fori_loop(0, zr * vecs_per_row, zloop, 0)
    for k in range(rows_per_tile // zr):
        pltpu.sync_copy(rows_v.at[0],
                        acc_s.at[pl.ds(sid * rows_per_tile + k * zr, zr)])
    plsc.subcore_barrier()

    rows0 = rows_v.at[0]
    rows1 = rows_v.at[1]

    def start(j, buf, idxbuf, sem):
        pltpu.async_copy(nf_hbm.at[src_v.at[j]], buf, sem)
        pltpu.async_copy(dstr_hbm.at[wid, j, 0], idxbuf, sem)

    def finish(j, buf, idxbuf, sem):
        pltpu.make_async_copy(nf_hbm.at[src_v.at[j]], buf, sem).wait()
        pltpu.make_async_copy(dstr_hbm.at[wid, j, 0], idxbuf, sem).wait()
        pltpu.sync_copy(buf, acc_s.at[idxbuf], add=True)

    start(0, rows0, dst_c.at[0], gsem0)

    def pair(jj, c):
        j0 = 2 * jj
        start(j0 + 1, rows1, dst_c.at[1], gsem1)
        finish(j0, rows0, dst_c.at[0], gsem0)
        start(j0 + 2, rows0, dst_c.at[0], gsem0)
        finish(j0 + 1, rows1, dst_c.at[1], gsem1)
        return c

    lax.fori_loop(0, (nch - 1) // 2, pair, 0)
    finish(nch - 1, rows0, dst_c.at[0], gsem0)
    plsc.subcore_barrier()
    for k in range(rows_per_tile // zr):
        sl = pl.ds(sid * rows_per_tile + k * zr, zr)
        pltpu.sync_copy(acc_s.at[sl], acc_hbm.at[cid, sl])


def _message_pass(nf, srcr, dstr, npad):
    n, d = nf.shape
    nch = srcr.shape[1]
    assert nch % 2 == 1
    f = pl.kernel(
        _mp_body,
        out_type=jax.ShapeDtypeStruct((NC, npad, d), jnp.float32),
        mesh=plsc.VectorSubcoreMesh(core_axis_name="c", subcore_axis_name="s"),
        compiler_params=pltpu.CompilerParams(needs_layout_passes=False),
        scratch_types=[
            pltpu.VMEM((nch, CH), jnp.int32),
            pltpu.VMEM((2, CH), jnp.int32),
            pltpu.VMEM((2, CH, d), jnp.float32),
            pltpu.SemaphoreType.DMA,
            pltpu.SemaphoreType.DMA,
            pltpu.VMEM_SHARED((npad, d), jnp.float32),
        ],
    )
    return f(nf, srcr, dstr)


def _final_body(acc_ref, nd_ref, out_ref):
    out_ref[...] = (acc_ref[0] + acc_ref[1]) * nd_ref[...]


def kernel(features, edge_index):
    n, d = features.shape
    e = edge_index.shape[1]
    assert e % (NW * CH) == 0 and d % L == 0
    src = edge_index[0].astype(jnp.int32)
    dst = edge_index[1].astype(jnp.int32)

    rpt = ((n + NS - 1) // NS + 2 * CH - 1) // (2 * CH) * (2 * CH)
    npad = NS * rpt

    nf, norm_dst = _degree_norms(src, dst, features, npad)

    r = 1000
    ep = e // NW
    srcr = src.reshape(NW, ep // CH, CH)
    dstr = dst.reshape(NW, ep // CH, 1, CH)
    acc = _message_pass(nf, srcr, dstr, npad)

    out = pl.pallas_call(
        _final_body,
        grid=(n // r,),
        in_specs=[
            pl.BlockSpec((NC, r, d), lambda i: (0, i, 0)),
            pl.BlockSpec((r, d), lambda i: (i, 0)),
        ],
        out_specs=pl.BlockSpec((r, d), lambda i: (i, 0)),
        out_shape=jax.ShapeDtypeStruct((n, d), jnp.float32),
    )(acc, norm_dst)
    return out
